# baseline (speedup 1.0000x reference)
import jax
import jax.numpy as jnp
from jax.experimental import pallas as pl
from jax.experimental.pallas import tpu as pltpu

B, S, D, E = 4, 2048, 4096, 64
S_CHUNK = 128
NS = S // S_CHUNK


def _router_body(lab_ref, h_ref, w_ref, logits_ref, probs_ref, loss_ref,
                 acc_ref):
    i = pl.program_id(0)

    @pl.when(i == 0)
    def _init():
        acc_ref[...] = jnp.zeros_like(acc_ref)

    csum = jnp.sum(h_ref[...], axis=1)
    acc_ref[...] += jax.lax.dot_general(
        csum, w_ref[...], (((1,), (1,)), ((), ())),
        preferred_element_type=jnp.float32)

    @pl.when(i == NS - 1)
    def _epilogue():
        logits = acc_ref[...] * (1.0 / S)
        m = jnp.max(logits, axis=1, keepdims=True)
        ex = jnp.exp(logits - m)
        se = jnp.sum(ex, axis=1, keepdims=True)
        logits_ref[...] = logits
        probs_ref[...] = ex / se
        lse = jnp.log(se) + m
        labcol = jnp.concatenate(
            [jnp.full((1, E), lab_ref[0, b], jnp.int32) for b in range(B)],
            axis=0)
        onehot = (labcol == jax.lax.broadcasted_iota(jnp.int32, (B, E), 1))
        picked = jnp.sum(jnp.where(onehot, logits, 0.0), axis=1,
                         keepdims=True)
        loss_ref[...] = jnp.mean(lse - picked).reshape(1, 1)


@jax.jit
def kernel(hidden_states, W, task_labels):
    logits, probs, loss = pl.pallas_call(
        _router_body,
        grid=(NS,),
        in_specs=[
            pl.BlockSpec(memory_space=pltpu.SMEM),
            pl.BlockSpec((B, S_CHUNK, D), lambda i: (0, i, 0)),
            pl.BlockSpec((E, D), lambda i: (0, 0)),
        ],
        out_specs=[
            pl.BlockSpec((B, E), lambda i: (0, 0)),
            pl.BlockSpec((B, E), lambda i: (0, 0)),
            pl.BlockSpec((1, 1), lambda i: (0, 0)),
        ],
        out_shape=[
            jax.ShapeDtypeStruct((B, E), jnp.float32),
            jax.ShapeDtypeStruct((B, E), jnp.float32),
            jax.ShapeDtypeStruct((1, 1), jnp.float32),
        ],
        scratch_shapes=[pltpu.VMEM((B, E), jnp.float32)],
    )(task_labels.reshape(1, B), hidden_states, W)
    return logits, probs, loss.reshape(())


# speedup vs baseline: 1.0130x; 1.0130x over previous
import jax
import jax.numpy as jnp
from jax.experimental import pallas as pl
from jax.experimental.pallas import tpu as pltpu

B, S, D, E = 4, 2048, 4096, 64
S_CHUNK = 128
NS = S // S_CHUNK


def _router_body(lab_ref, h_ref, w_ref, logits_ref, probs_ref, loss_ref,
                 acc_ref):
    i = pl.program_id(0)

    @pl.when(i == 0)
    def _init():
        acc_ref[...] = jnp.zeros_like(acc_ref)

    acc_ref[...] += jnp.sum(h_ref[...], axis=1)

    @pl.when(i == NS - 1)
    def _epilogue():
        pooled = acc_ref[...] * (1.0 / S)
        logits = jax.lax.dot_general(
            pooled, w_ref[...], (((1,), (1,)), ((), ())),
            preferred_element_type=jnp.float32)
        m = jnp.max(logits, axis=1, keepdims=True)
        ex = jnp.exp(logits - m)
        se = jnp.sum(ex, axis=1, keepdims=True)
        logits_ref[...] = logits
        probs_ref[...] = ex / se
        lse = jnp.log(se) + m
        labcol = jnp.concatenate(
            [jnp.full((1, E), lab_ref[0, b], jnp.int32) for b in range(B)],
            axis=0)
        onehot = (labcol == jax.lax.broadcasted_iota(jnp.int32, (B, E), 1))
        picked = jnp.sum(jnp.where(onehot, logits, 0.0), axis=1,
                         keepdims=True)
        loss_ref[...] = jnp.mean(lse - picked).reshape(1, 1)


@jax.jit
def kernel(hidden_states, W, task_labels):
    logits, probs, loss = pl.pallas_call(
        _router_body,
        grid=(NS,),
        in_specs=[
            pl.BlockSpec(memory_space=pltpu.SMEM),
            pl.BlockSpec((B, S_CHUNK, D), lambda i: (0, i, 0)),
            pl.BlockSpec((E, D), lambda i: (0, 0)),
        ],
        out_specs=[
            pl.BlockSpec((B, E), lambda i: (0, 0)),
            pl.BlockSpec((B, E), lambda i: (0, 0)),
            pl.BlockSpec((1, 1), lambda i: (0, 0)),
        ],
        out_shape=[
            jax.ShapeDtypeStruct((B, E), jnp.float32),
            jax.ShapeDtypeStruct((B, E), jnp.float32),
            jax.ShapeDtypeStruct((1, 1), jnp.float32),
        ],
        scratch_shapes=[pltpu.VMEM((B, D), jnp.float32)],
    )(task_labels.reshape(1, B), hidden_states, W)
    return logits, probs, loss.reshape(())
